# Initial kernel scaffold; baseline (speedup 1.0000x reference)
import jax
import jax.numpy as jnp
from jax.experimental import pallas as pl


def kernel(x, w0, b0, w1, b1, w2, b2, w3, b3, w4, b4, w5, b5, w6, b6, w7, b7, w8, b8, w9, b9):
    raise NotImplementedError("write your pallas kernel here")


# speedup vs baseline: 1.3489x; 1.3489x over previous
import functools

import jax
import jax.numpy as jnp
from jax.experimental import pallas as pl
from jax.experimental.pallas import tpu as pltpu


def _conv_body(x_ref, w_ref, b_ref, *out_refs, hb, width, cin3, cob,
               pool, emit_full):
    rb = pl.program_id(2)
    r0 = rb * hb
    acc = jnp.zeros((hb * width, cob), jnp.float32)
    for dy in range(3):
        patch = x_ref[0, pl.ds(r0 + dy, hb), :, :]
        acc = acc + jnp.dot(patch.reshape(hb * width, cin3), w_ref[dy],
                            preferred_element_type=jnp.float32)
    y = jnp.maximum(acc + b_ref[...], 0.0)
    i = 0
    if emit_full:
        out_refs[i][0] = y.reshape(hb, width, cob)
        i += 1
    if pool:
        m = jnp.max(y.reshape(hb // 2, 2, width, cob), axis=1)
        p = jnp.max(m.reshape(hb // 2, width // 2, 2, cob), axis=2)
        out_refs[i][0] = p.astype(jnp.bfloat16)
    elif not emit_full:
        out_refs[i][0] = y.reshape(hb, width, cob).astype(jnp.bfloat16)


def _conv_layer(xs, w3, b2, *, pool, emit_full):
    n, hp, width, cin3 = xs.shape
    h = hp - 2
    cout = w3.shape[-1]
    cob = cout if cout <= 256 else 256
    hb = min(h, max(2, (131072 // cob) // width))

    out_shapes = []
    out_specs = []
    if emit_full:
        out_shapes.append(jax.ShapeDtypeStruct((n, h, width, cout), jnp.float32))
        out_specs.append(pl.BlockSpec((1, hb, width, cob),
                                      lambda co, ni, rb: (ni, rb, 0, co)))
    if pool:
        out_shapes.append(jax.ShapeDtypeStruct((n, h // 2, width // 2, cout),
                                               jnp.bfloat16))
        out_specs.append(pl.BlockSpec((1, hb // 2, width // 2, cob),
                                      lambda co, ni, rb: (ni, rb, 0, co)))
    elif not emit_full:
        out_shapes.append(jax.ShapeDtypeStruct((n, h, width, cout), jnp.bfloat16))
        out_specs.append(pl.BlockSpec((1, hb, width, cob),
                                      lambda co, ni, rb: (ni, rb, 0, co)))

    body = functools.partial(_conv_body, hb=hb, width=width, cin3=cin3, cob=cob,
                             pool=pool, emit_full=emit_full)
    outs = pl.pallas_call(
        body,
        out_shape=out_shapes,
        grid=(cout // cob, n, h // hb),
        in_specs=[
            pl.BlockSpec((1, hp, width, cin3), lambda co, ni, rb: (ni, 0, 0, 0)),
            pl.BlockSpec((3, cin3, cob), lambda co, ni, rb: (0, 0, co)),
            pl.BlockSpec((1, cob), lambda co, ni, rb: (0, co)),
        ],
        out_specs=out_specs,
        compiler_params=pltpu.CompilerParams(
            dimension_semantics=("parallel", "parallel", "parallel")),
    )(xs, w3, b2)
    return outs


def _shift3(a):
    xp = jnp.pad(a, ((0, 0), (1, 1), (1, 1), (0, 0)))
    w = a.shape[2]
    s = jnp.stack([xp[:, :, dx:dx + w, :] for dx in range(3)], axis=3)
    return s.reshape(a.shape[0], a.shape[1] + 2, w, 3 * a.shape[3])


def kernel(x, w0, b0, w1, b1, w2, b2, w3, b3, w4, b4,
           w5, b5, w6, b6, w7, b7, w8, b8, w9, b9):
    ws = [w0, w1, w2, w3, w4, w5, w6, w7, w8, w9]
    bs = [b0, b1, b2, b3, b4, b5, b6, b7, b8, b9]
    w3s = [w.reshape(3, 3 * w.shape[2], w.shape[3]).astype(jnp.bfloat16)
           for w in ws]
    b2s = [b.reshape(1, -1) for b in bs]

    h = jnp.transpose(x, (0, 2, 3, 1)).astype(jnp.bfloat16)

    (h,) = _conv_layer(_shift3(h), w3s[0], b2s[0], pool=False, emit_full=False)
    r1, h = _conv_layer(_shift3(h), w3s[1], b2s[1], pool=True, emit_full=True)
    (h,) = _conv_layer(_shift3(h), w3s[2], b2s[2], pool=False, emit_full=False)
    r2, h = _conv_layer(_shift3(h), w3s[3], b2s[3], pool=True, emit_full=True)
    (h,) = _conv_layer(_shift3(h), w3s[4], b2s[4], pool=False, emit_full=False)
    (h,) = _conv_layer(_shift3(h), w3s[5], b2s[5], pool=False, emit_full=False)
    r3, h = _conv_layer(_shift3(h), w3s[6], b2s[6], pool=True, emit_full=True)
    (h,) = _conv_layer(_shift3(h), w3s[7], b2s[7], pool=False, emit_full=False)
    (h,) = _conv_layer(_shift3(h), w3s[8], b2s[8], pool=False, emit_full=False)
    (r4,) = _conv_layer(_shift3(h), w3s[9], b2s[9], pool=False, emit_full=True)

    to_nchw = lambda t: jnp.transpose(t, (0, 3, 1, 2))
    return tuple(to_nchw(t) for t in (r1, r2, r3, r4))


# speedup vs baseline: 2.6242x; 1.9454x over previous
import functools

import jax
import jax.numpy as jnp
from jax.experimental import pallas as pl
from jax.experimental.pallas import tpu as pltpu


def _stack_store(o_ref, z, r0z, zhb, zh, cob, rb, nrb):
    zw = z.shape[1]
    zb = z.astype(jnp.bfloat16)
    zero_col = jnp.zeros((zhb, 1, cob), jnp.bfloat16)
    sl = jnp.concatenate([zero_col, zb[:, :zw - 1, :]], axis=1)
    sr = jnp.concatenate([zb[:, 1:, :], zero_col], axis=1)
    o_ref[0, pl.ds(r0z + 1, zhb), :, 0 * cob:1 * cob] = sl
    o_ref[0, pl.ds(r0z + 1, zhb), :, 1 * cob:2 * cob] = zb
    o_ref[0, pl.ds(r0z + 1, zhb), :, 2 * cob:3 * cob] = sr

    @pl.when(rb == 0)
    def _():
        o_ref[0, 0] = jnp.zeros((zw, 3 * cob), jnp.bfloat16)

    @pl.when(rb == nrb - 1)
    def _():
        o_ref[0, zh + 1] = jnp.zeros((zw, 3 * cob), jnp.bfloat16)


def _conv_body(x_ref, w_ref, b_ref, *out_refs, hb, width, h, cin3, cob,
               pool, emit_full, emit_stacked):
    rb = pl.program_id(2)
    nrb = pl.num_programs(2)
    r0 = rb * hb
    acc = jnp.zeros((hb * width, cob), jnp.float32)
    for dy in range(3):
        patch = x_ref[0, pl.ds(r0 + dy, hb), :, :]
        acc = acc + jnp.dot(patch.reshape(hb * width, cin3), w_ref[dy],
                            preferred_element_type=jnp.float32)
    y = jnp.maximum(acc + b_ref[...], 0.0)
    i = 0
    if emit_full:
        out_refs[i][0] = y.reshape(hb, width, cob)
        i += 1
    if emit_stacked:
        if pool:
            m = jnp.max(y.reshape(hb // 2, 2, width, cob), axis=1)
            z = jnp.max(m.reshape(hb // 2, width // 2, 2, cob), axis=2)
            _stack_store(out_refs[i], z, rb * (hb // 2), hb // 2, h // 2,
                         cob, rb, nrb)
        else:
            _stack_store(out_refs[i], y.reshape(hb, width, cob), r0, hb, h,
                         cob, rb, nrb)


def _conv_layer(xs, w3, b2, *, pool, emit_full, emit_stacked=True):
    n, hp, width, cin3 = xs.shape
    h = hp - 2
    cout = w3.shape[-1]
    cob = cout if cout <= 256 else 256
    hb = min(h, max(2, (131072 // cob) // width))

    out_shapes = []
    out_specs = []
    if emit_full:
        out_shapes.append(jax.ShapeDtypeStruct((n, h, width, cout), jnp.float32))
        out_specs.append(pl.BlockSpec((1, hb, width, cob),
                                      lambda co, ni, rb: (ni, rb, 0, co)))
    if emit_stacked:
        zh, zw = (h // 2, width // 2) if pool else (h, width)
        out_shapes.append(jax.ShapeDtypeStruct((n, zh + 2, zw, 3 * cout),
                                               jnp.bfloat16))
        out_specs.append(pl.BlockSpec((1, zh + 2, zw, 3 * cob),
                                      lambda co, ni, rb: (ni, 0, 0, co)))

    body = functools.partial(_conv_body, hb=hb, width=width, h=h, cin3=cin3,
                             cob=cob, pool=pool, emit_full=emit_full,
                             emit_stacked=emit_stacked)
    outs = pl.pallas_call(
        body,
        out_shape=out_shapes,
        grid=(cout // cob, n, h // hb),
        in_specs=[
            pl.BlockSpec((1, hp, width, cin3), lambda co, ni, rb: (ni, 0, 0, 0)),
            pl.BlockSpec((3, cin3, cob), lambda co, ni, rb: (0, 0, co)),
            pl.BlockSpec((1, cob), lambda co, ni, rb: (0, co)),
        ],
        out_specs=out_specs,
        compiler_params=pltpu.CompilerParams(
            dimension_semantics=("parallel", "parallel", "arbitrary")),
    )(xs, w3, b2)
    return outs


def _shift3(a):
    xp = jnp.pad(a, ((0, 0), (1, 1), (1, 1), (0, 0)))
    w = a.shape[2]
    s = jnp.stack([xp[:, :, dx:dx + w, :] for dx in range(3)], axis=3)
    return s.reshape(a.shape[0], a.shape[1] + 2, w, 3 * a.shape[3])


def _w3(w, groups):
    cin, cout = w.shape[2], w.shape[3]
    wb = w.astype(jnp.bfloat16)
    if groups == 1:
        return wb.reshape(3, 3 * cin, cout)
    gs = cin // groups
    return (wb.reshape(3, 3, groups, gs, cout)
            .transpose(0, 2, 1, 3, 4).reshape(3, 3 * cin, cout))


def kernel(x, w0, b0, w1, b1, w2, b2, w3, b3, w4, b4,
           w5, b5, w6, b6, w7, b7, w8, b8, w9, b9):
    ws = [w0, w1, w2, w3, w4, w5, w6, w7, w8, w9]
    bs = [b0, b1, b2, b3, b4, b5, b6, b7, b8, b9]
    groups = [1, 1, 1, 1, 1, 1, 1, 1, 2, 2]
    w3s = [_w3(w, g) for w, g in zip(ws, groups)]
    b2s = [b.reshape(1, -1) for b in bs]

    h = jnp.transpose(x, (0, 2, 3, 1)).astype(jnp.bfloat16)

    (h,) = _conv_layer(_shift3(h), w3s[0], b2s[0], pool=False, emit_full=False)
    r1, h = _conv_layer(h, w3s[1], b2s[1], pool=True, emit_full=True)
    (h,) = _conv_layer(h, w3s[2], b2s[2], pool=False, emit_full=False)
    r2, h = _conv_layer(h, w3s[3], b2s[3], pool=True, emit_full=True)
    (h,) = _conv_layer(h, w3s[4], b2s[4], pool=False, emit_full=False)
    (h,) = _conv_layer(h, w3s[5], b2s[5], pool=False, emit_full=False)
    r3, h = _conv_layer(h, w3s[6], b2s[6], pool=True, emit_full=True)
    (h,) = _conv_layer(h, w3s[7], b2s[7], pool=False, emit_full=False)
    (h,) = _conv_layer(h, w3s[8], b2s[8], pool=False, emit_full=False)
    (r4,) = _conv_layer(h, w3s[9], b2s[9], pool=False, emit_full=True,
                        emit_stacked=False)

    to_nchw = lambda t: jnp.transpose(t, (0, 3, 1, 2))
    return tuple(to_nchw(t) for t in (r1, r2, r3, r4))


# speedup vs baseline: 3.1233x; 1.1902x over previous
import functools

import jax
import jax.numpy as jnp
from jax.experimental import pallas as pl
from jax.experimental.pallas import tpu as pltpu


def _stack_store(o_ref, z, r0z, zhb, zh, cob, rb, nrb, rpad):
    zw = z.shape[1]
    zb = z.astype(jnp.bfloat16)
    zero_col = jnp.zeros((zhb, 1, cob), jnp.bfloat16)
    sl = jnp.concatenate([zero_col, zb[:, :zw - 1, :]], axis=1)
    sr = jnp.concatenate([zb[:, 1:, :], zero_col], axis=1)
    o_ref[0, pl.ds(r0z + rpad, zhb), :, :] = jnp.concatenate(
        [sl, zb, sr], axis=2)

    @pl.when(rb == 0)
    def _():
        o_ref[0, 0:rpad] = jnp.zeros((rpad, zw, 3 * cob), jnp.bfloat16)

    @pl.when(rb == nrb - 1)
    def _():
        o_ref[0, zh + rpad:zh + 2 * rpad] = jnp.zeros((rpad, zw, 3 * cob),
                                                      jnp.bfloat16)


def _emit(out_refs, y, *, hb, width, h, cob, pool, emit_full, out_rpad,
          rb, nrb):
    i = 0
    if emit_full:
        out_refs[i][0] = y.reshape(hb, width, cob)
        i += 1
    if i < len(out_refs):
        if pool:
            m = jnp.max(y.reshape(hb // 2, 2, width, cob), axis=1)
            z = jnp.max(m.reshape(hb // 2, width // 2, 2, cob), axis=2)
            _stack_store(out_refs[i], z, rb * (hb // 2), hb // 2, h // 2,
                         cob, rb, nrb, out_rpad)
        else:
            _stack_store(out_refs[i], y.reshape(hb, width, cob), rb * hb,
                         hb, h, cob, rb, nrb, out_rpad)


def _stacked_out(n, h, width, cout, *, pool, out_rpad):
    zh, zw = (h // 2, width // 2) if pool else (h, width)
    shape = jax.ShapeDtypeStruct((n, zh + 2 * out_rpad, zw, 3 * cout),
                                 jnp.bfloat16)
    spec = pl.BlockSpec((1, zh + 2 * out_rpad, zw, 3 * cout),
                        lambda co, ni, rb: (ni, 0, 0, 0))
    return shape, spec


def _conv_body(x_ref, w_ref, b_ref, *out_refs, hb, width, h, cin3, cob,
               pool, emit_full, out_rpad):
    rb = pl.program_id(2)
    nrb = pl.num_programs(2)
    r0 = rb * hb
    acc = jnp.zeros((hb * width, cob), jnp.float32)
    for dy in range(3):
        patch = x_ref[0, pl.ds(r0 + dy, hb), :, :]
        acc = acc + jnp.dot(patch.reshape(hb * width, cin3), w_ref[dy],
                            preferred_element_type=jnp.float32)
    y = jnp.maximum(acc + b_ref[...], 0.0)
    _emit(out_refs, y, hb=hb, width=width, h=h, cob=cob, pool=pool,
          emit_full=emit_full, out_rpad=out_rpad, rb=rb, nrb=nrb)


def _conv_layer(xs, w3, b2, *, pool, emit_full, emit_stacked=True,
                out_rpad=1):
    n, hp, width, cin3 = xs.shape
    h = hp - 2
    cout = w3.shape[-1]
    cob = cout if cout <= 256 else 256
    hb = min(h, max(2, (131072 // cob) // width))

    out_shapes = []
    out_specs = []
    if emit_full:
        out_shapes.append(jax.ShapeDtypeStruct((n, h, width, cout), jnp.float32))
        out_specs.append(pl.BlockSpec((1, hb, width, cob),
                                      lambda co, ni, rb: (ni, rb, 0, co)))
    if emit_stacked:
        shape, spec = _stacked_out(n, h, width, cout, pool=pool,
                                   out_rpad=out_rpad)
        out_shapes.append(shape)
        out_specs.append(spec)

    body = functools.partial(_conv_body, hb=hb, width=width, h=h, cin3=cin3,
                             cob=cob, pool=pool, emit_full=emit_full,
                             out_rpad=out_rpad)
    return pl.pallas_call(
        body,
        out_shape=out_shapes,
        grid=(cout // cob, n, h // hb),
        in_specs=[
            pl.BlockSpec((1, hp, width, cin3), lambda co, ni, rb: (ni, 0, 0, 0)),
            pl.BlockSpec((3, cin3, cob), lambda co, ni, rb: (0, 0, co)),
            pl.BlockSpec((1, cob), lambda co, ni, rb: (0, co)),
        ],
        out_specs=out_specs,
        compiler_params=pltpu.CompilerParams(
            dimension_semantics=("parallel", "parallel", "arbitrary")),
    )(xs, w3, b2)


def _conv2_body(x_ref, wa_ref, ba_ref, wb_ref, bb_ref, *out_refs,
                hb, width, h, cin3, ca, cob, pool, emit_full, out_rpad):
    rb = pl.program_id(2)
    nrb = pl.num_programs(2)
    r0 = rb * hb
    ma = hb + 2
    acca = jnp.zeros((ma * width, ca), jnp.float32)
    for dy in range(3):
        patch = x_ref[0, pl.ds(r0 + dy, ma), :, :]
        acca = acca + jnp.dot(patch.reshape(ma * width, cin3), wa_ref[dy],
                              preferred_element_type=jnp.float32)
    ya = jnp.maximum(acca + ba_ref[...], 0.0).reshape(ma, width, ca)
    gr = r0 - 1 + jax.lax.broadcasted_iota(jnp.int32, (ma, 1, 1), 0)
    ya = jnp.where((gr >= 0) & (gr < h), ya, 0.0).astype(jnp.bfloat16)
    zero_col = jnp.zeros((ma, 1, ca), jnp.bfloat16)
    sl = jnp.concatenate([zero_col, ya[:, :width - 1, :]], axis=1)
    sr = jnp.concatenate([ya[:, 1:, :], zero_col], axis=1)
    stacked = jnp.concatenate([sl, ya, sr], axis=2)
    accb = jnp.zeros((hb * width, cob), jnp.float32)
    for dy in range(3):
        patch = stacked[dy:dy + hb]
        accb = accb + jnp.dot(patch.reshape(hb * width, 3 * ca), wb_ref[dy],
                              preferred_element_type=jnp.float32)
    y = jnp.maximum(accb + bb_ref[...], 0.0)
    _emit(out_refs, y, hb=hb, width=width, h=h, cob=cob, pool=pool,
          emit_full=emit_full, out_rpad=out_rpad, rb=rb, nrb=nrb)


def _conv2_layer(xs, wa3, ba2, wb3, bb2, *, hb, pool, emit_full, out_rpad):
    n, hp, width, cin3 = xs.shape
    h = hp - 4
    ca = wa3.shape[-1]
    cob = wb3.shape[-1]

    out_shapes = []
    out_specs = []
    if emit_full:
        out_shapes.append(jax.ShapeDtypeStruct((n, h, width, cob), jnp.float32))
        out_specs.append(pl.BlockSpec((1, hb, width, cob),
                                      lambda co, ni, rb: (ni, rb, 0, 0)))
    shape, spec = _stacked_out(n, h, width, cob, pool=pool, out_rpad=out_rpad)
    out_shapes.append(shape)
    out_specs.append(spec)

    body = functools.partial(_conv2_body, hb=hb, width=width, h=h, cin3=cin3,
                             ca=ca, cob=cob, pool=pool, emit_full=emit_full,
                             out_rpad=out_rpad)
    return pl.pallas_call(
        body,
        out_shape=out_shapes,
        grid=(1, n, h // hb),
        in_specs=[
            pl.BlockSpec((1, hp, width, cin3), lambda co, ni, rb: (ni, 0, 0, 0)),
            pl.BlockSpec((3, cin3, ca), lambda co, ni, rb: (0, 0, 0)),
            pl.BlockSpec((1, ca), lambda co, ni, rb: (0, 0)),
            pl.BlockSpec((3, 3 * ca, cob), lambda co, ni, rb: (0, 0, 0)),
            pl.BlockSpec((1, cob), lambda co, ni, rb: (0, 0)),
        ],
        out_specs=out_specs,
        compiler_params=pltpu.CompilerParams(
            dimension_semantics=("parallel", "parallel", "arbitrary")),
    )(xs, wa3, ba2, wb3, bb2)


def _shift3(a, rpad=1):
    xp = jnp.pad(a, ((0, 0), (rpad, rpad), (1, 1), (0, 0)))
    w = a.shape[2]
    s = jnp.stack([xp[:, :, dx:dx + w, :] for dx in range(3)], axis=3)
    return s.reshape(a.shape[0], a.shape[1] + 2 * rpad, w, 3 * a.shape[3])


def _w3(w):
    return w.reshape(3, 3 * w.shape[2], w.shape[3]).astype(jnp.bfloat16)


def kernel(x, w0, b0, w1, b1, w2, b2, w3, b3, w4, b4,
           w5, b5, w6, b6, w7, b7, w8, b8, w9, b9):
    ws = [w0, w1, w2, w3, w4, w5, w6, w7, w8, w9]
    bs = [b0, b1, b2, b3, b4, b5, b6, b7, b8, b9]
    w3s = [_w3(w) for w in ws]
    b2s = [b.reshape(1, -1) for b in bs]

    h = jnp.transpose(x, (0, 2, 3, 1)).astype(jnp.bfloat16)

    r1, h = _conv2_layer(_shift3(h, rpad=2), w3s[0], b2s[0], w3s[1], b2s[1],
                         hb=16, pool=True, emit_full=True, out_rpad=2)
    r2, h = _conv2_layer(h, w3s[2], b2s[2], w3s[3], b2s[3],
                         hb=16, pool=True, emit_full=True, out_rpad=2)
    (h,) = _conv2_layer(h, w3s[4], b2s[4], w3s[5], b2s[5],
                        hb=16, pool=False, emit_full=False, out_rpad=1)
    r3, h = _conv_layer(h, w3s[6], b2s[6], pool=True, emit_full=True,
                        out_rpad=2)
    (h,) = _conv2_layer(h, w3s[7], b2s[7], w3s[8], b2s[8],
                        hb=16, pool=False, emit_full=False, out_rpad=1)
    (r4,) = _conv_layer(h, w3s[9], b2s[9], pool=False, emit_full=True,
                        emit_stacked=False)

    to_nchw = lambda t: jnp.transpose(t, (0, 3, 1, 2))
    return tuple(to_nchw(t) for t in (r1, r2, r3, r4))
